# Initial kernel scaffold; baseline (speedup 1.0000x reference)
import jax
import jax.numpy as jnp
from jax.experimental import pallas as pl


def kernel(input, weight, bias, indx_seqs):
    raise NotImplementedError("write your pallas kernel here")


# speedup vs baseline: 14.1773x; 14.1773x over previous
import functools

import jax
import jax.numpy as jnp
from jax.experimental import pallas as pl
import jax.experimental.pallas.tpu as pltpu


def _blk_kernel(idx_ref, w_ref, x_ref, b_ref, out_ref, *, in_features, bo):
    fan = idx_ref.shape[0]
    iota = jax.lax.broadcasted_iota(jnp.int32, (in_features, bo), 0)
    idx = idx_ref[...]
    w = w_ref[...]
    s = jnp.zeros((in_features, bo), jnp.float32)
    for f in range(fan):
        s = s + jnp.where(iota == idx[f : f + 1, :], w[f : f + 1, :], 0.0)
    out_ref[...] = (
        jnp.dot(x_ref[...], s, preferred_element_type=jnp.float32)
        + b_ref[...]
    )


def kernel(input, weight, bias, indx_seqs):
    batch, in_features = input.shape
    out_features, fan_in = weight.shape
    bo = min(256, out_features)
    n_blk = out_features // bo

    idx_t = indx_seqs.astype(jnp.int32).T
    w_t = weight.T
    bias2 = bias.reshape(1, out_features)

    grid = (n_blk,)
    out = pl.pallas_call(
        functools.partial(_blk_kernel, in_features=in_features, bo=bo),
        grid=grid,
        in_specs=[
            pl.BlockSpec((fan_in, bo), lambda j: (0, j)),
            pl.BlockSpec((fan_in, bo), lambda j: (0, j)),
            pl.BlockSpec((batch, in_features), lambda j: (0, 0)),
            pl.BlockSpec((1, bo), lambda j: (0, j)),
        ],
        out_specs=pl.BlockSpec((batch, bo), lambda j: (0, j)),
        out_shape=jax.ShapeDtypeStruct((batch, out_features), jnp.float32),
    )(idx_t, w_t, input, bias2)
    return out
